# Initial kernel scaffold; baseline (speedup 1.0000x reference)
import jax
import jax.numpy as jnp
from jax.experimental import pallas as pl


def kernel(w1p, b1c, w2p, b2c, wfc_canon, bfc_row, gmask, bdmask, fold, x_nchw):
    raise NotImplementedError("write your pallas kernel here")


# speedup vs baseline: 4.9606x; 4.9606x over previous
import numpy as np

import jax
import jax.numpy as jnp
from jax.experimental import pallas as pl
from jax.experimental.pallas import tpu as pltpu

_S = 128
_H = 28
_C1, _C2, _NCLS = 10, 15, 10
_XROW = 32
_AROW = 320
_K1 = 6 * _XROW
_N1 = 2 * _H * _C1
_K2 = 2 * _AROW + 30 * _C1
_N2 = _H * _C2
_KF = _H * _H * _C2


def _conv1_tables():
    dr = np.arange(6)[:, None, None, None, None]
    dwcol = np.arange(_XROW)[None, :, None, None, None]
    r = np.arange(2)[None, None, :, None, None]
    w = np.arange(_H)[None, None, None, :, None]
    c = np.arange(_C1)[None, None, None, None, :]
    kh = dr - r
    kw = dwcol - w
    valid = (kh >= 0) & (kh < 5) & (kw >= 0) & (kw < 5)
    tap = np.clip(kh, 0, 4) * 5 + np.clip(kw, 0, 4)
    flat = c * 25 + tap
    flat = np.broadcast_to(flat, (6, _XROW, 2, _H, _C1)).reshape(_K1, _N1)
    mask = np.broadcast_to(valid, (6, _XROW, 2, _H, _C1)).reshape(_K1, _N1)
    return flat.astype(np.int32), mask.astype(np.float32)


def _conv2_tables():
    dr = np.arange(3)[:, None, None, None, None]
    pcol = np.arange(_XROW)[None, :, None, None, None]
    c1 = np.arange(_C1)[None, None, :, None, None]
    w = np.arange(_H)[None, None, None, :, None]
    c2 = np.arange(_C2)[None, None, None, None, :]
    kw = pcol - w
    valid = (kw >= 0) & (kw < 3)
    flat = c2 * 90 + (dr * 3 + np.clip(kw, 0, 2)) * _C1 + c1
    flat = np.broadcast_to(flat, (3, _XROW, _C1, _H, _C2))
    flat = flat.reshape(3 * _XROW * _C1, _N2)[:_K2]
    mask = np.broadcast_to(valid, (3, _XROW, _C1, _H, _C2))
    mask = mask.reshape(3 * _XROW * _C1, _N2)[:_K2]
    return flat.astype(np.int32), mask.astype(np.float32)


_W1_IDX, _W1_MASK = _conv1_tables()
_W2_IDX, _W2_MASK = _conv2_tables()


def _fused_body(x_ref, w1_ref, b1_ref, w2_ref, b2_ref, wfc_ref, bfc_ref,
                o_ref, xp_ref, a1_ref, a2_ref):
    f32 = jnp.float32

    xp_ref[...] = jnp.zeros((_S, _XROW * _XROW), f32)
    for r in range(_H):
        base = _XROW * (r + 2) + 2
        xp_ref[:, base:base + _H] = x_ref[:, _H * r:_H * (r + 1)]

    a1_ref[...] = jnp.zeros((_S, 30 * _AROW), f32)

    b1v = b1_ref[...]
    for h in range(0, _H, 2):
        t = jnp.dot(xp_ref[:, _XROW * h:_XROW * h + _K1], w1_ref[...],
                    preferred_element_type=f32)
        t = jnp.maximum(t + b1v, 0.0)
        a1_ref[:, _AROW * (h + 1) + 10:_AROW * (h + 1) + 290] = t[:, :280]
        a1_ref[:, _AROW * (h + 2) + 10:_AROW * (h + 2) + 290] = t[:, 280:]

    b2v = b2_ref[...]
    for h in range(_H):
        t = jnp.dot(a1_ref[:, _AROW * h:_AROW * h + _K2], w2_ref[...],
                    preferred_element_type=f32)
        a2_ref[:, _N2 * h:_N2 * (h + 1)] = jnp.maximum(t + b2v, 0.0)

    logits = jnp.dot(a2_ref[...], wfc_ref[...], preferred_element_type=f32)
    o_ref[...] = (logits + bfc_ref[...]).astype(o_ref.dtype)


def kernel(w1p, b1c, w2p, b2c, wfc_canon, bfc_row, gmask, bdmask, fold, x_nchw):
    del gmask, bdmask, fold
    f32 = jnp.float32
    n = x_nchw.shape[0]
    x = x_nchw.astype(f32).reshape(n, _H * _H)
    n_pad = pl.cdiv(n, _S) * _S
    if n_pad != n:
        x = jnp.pad(x, ((0, n_pad - n), (0, 0)))

    w1r = w1p.astype(f32).reshape(-1)[_W1_IDX] * _W1_MASK
    b1row = jnp.tile(b1c.astype(f32)[:, 0], (2 * _H,)).reshape(1, _N1)
    w2r = w2p.astype(f32).reshape(-1)[_W2_IDX] * _W2_MASK
    b2row = jnp.tile(b2c.astype(f32)[:, 0], (_H,)).reshape(1, _N2)
    wfc = jnp.pad(wfc_canon.astype(f32), ((0, 4), (0, 0)))
    wfc = wfc.reshape(_H, _XROW, _C2, _NCLS)[:, :_H]
    wfc = wfc.reshape(_KF, _NCLS)

    out = pl.pallas_call(
        _fused_body,
        out_shape=jax.ShapeDtypeStruct((n_pad, _NCLS), f32),
        grid=(n_pad // _S,),
        in_specs=[
            pl.BlockSpec((_S, _H * _H), lambda i: (i, 0)),
            pl.BlockSpec((_K1, _N1), lambda i: (0, 0)),
            pl.BlockSpec((1, _N1), lambda i: (0, 0)),
            pl.BlockSpec((_K2, _N2), lambda i: (0, 0)),
            pl.BlockSpec((1, _N2), lambda i: (0, 0)),
            pl.BlockSpec((_KF, _NCLS), lambda i: (0, 0)),
            pl.BlockSpec((1, _NCLS), lambda i: (0, 0)),
        ],
        out_specs=pl.BlockSpec((_S, _NCLS), lambda i: (i, 0)),
        scratch_shapes=[
            pltpu.VMEM((_S, _XROW * _XROW), f32),
            pltpu.VMEM((_S, 30 * _AROW), f32),
            pltpu.VMEM((_S, _KF), f32),
        ],
        compiler_params=pltpu.CompilerParams(
            dimension_semantics=("parallel",),
        ),
    )(x, w1r, b1row, w2r, b2row, wfc, bfc_row.astype(f32))
    return out[:n]


# speedup vs baseline: 16.9194x; 3.4107x over previous
import numpy as np

import jax
import jax.numpy as jnp
from jax.experimental import pallas as pl
from jax.experimental.pallas import tpu as pltpu

_S = 128
_H = 28
_C1, _C2, _NCLS = 10, 15, 10
_XROW = 32
_AROW = 320
_K1 = 6 * _XROW
_N1 = 2 * _H * _C1
_K2 = 2 * _AROW + 30 * _C1
_N2 = _H * _C2
_KF = _H * _H * _C2


_D1R = (np.arange(6)[None, :, None] - np.arange(2)[None, None, :]
        == np.arange(5)[:, None, None]).astype(np.float32)
_D1W = (np.arange(_XROW)[None, :, None] - np.arange(_H)[None, None, :]
        == np.arange(5)[:, None, None]).astype(np.float32)
_D2W = (np.arange(_XROW)[None, :, None] - np.arange(_H)[None, None, :]
        == np.arange(3)[:, None, None]).astype(np.float32)


def _fused_body(x_ref, w1_ref, b1_ref, w2_ref, b2_ref, wfc_ref, bfc_ref,
                o_ref, xp_ref, a1_ref, a2_ref):
    f32 = jnp.float32

    xp_ref[...] = jnp.zeros((_S, _XROW * _XROW), f32)
    for r in range(_H):
        base = _XROW * (r + 2) + 2
        xp_ref[:, base:base + _H] = x_ref[:, _H * r:_H * (r + 1)]

    a1_ref[...] = jnp.zeros((_S, 30 * _AROW), f32)

    b1v = b1_ref[...]
    for h in range(0, _H, 2):
        t = jnp.dot(xp_ref[:, _XROW * h:_XROW * h + _K1], w1_ref[...],
                    preferred_element_type=f32)
        t = jnp.maximum(t + b1v, 0.0)
        a1_ref[:, _AROW * (h + 1) + 10:_AROW * (h + 1) + 290] = t[:, :280]
        a1_ref[:, _AROW * (h + 2) + 10:_AROW * (h + 2) + 290] = t[:, 280:]

    b2v = b2_ref[...]
    for h in range(_H):
        t = jnp.dot(a1_ref[:, _AROW * h:_AROW * h + _K2], w2_ref[...],
                    preferred_element_type=f32)
        a2_ref[:, _N2 * h:_N2 * (h + 1)] = jnp.maximum(t + b2v, 0.0)

    logits = jnp.dot(a2_ref[...], wfc_ref[...], preferred_element_type=f32)
    o_ref[...] = (logits + bfc_ref[...]).astype(o_ref.dtype)


def kernel(w1p, b1c, w2p, b2c, wfc_canon, bfc_row, gmask, bdmask, fold, x_nchw):
    del gmask, bdmask, fold
    f32 = jnp.float32
    n = x_nchw.shape[0]
    x = x_nchw.astype(f32).reshape(n, _H * _H)
    n_pad = pl.cdiv(n, _S) * _S
    if n_pad != n:
        x = jnp.pad(x, ((0, n_pad - n), (0, 0)))

    w1t = w1p.astype(f32).reshape(_C1, 5, 5)
    w1r = jnp.einsum('har,kbw,chk->abrwc', _D1R, _D1W, w1t)
    w1r = w1r.reshape(_K1, _N1)
    b1row = jnp.tile(b1c.astype(f32)[:, 0], (2 * _H,)).reshape(1, _N1)
    w2t = w2p.astype(f32).reshape(_C2, 3, 3, _C1)
    w2r = jnp.einsum('kpw,cdkb->dpbwc', _D2W, w2t)
    w2r = w2r.reshape(3 * _XROW * _C1, _N2)[:_K2]
    b2row = jnp.tile(b2c.astype(f32)[:, 0], (_H,)).reshape(1, _N2)
    wfc = jnp.pad(wfc_canon.astype(f32), ((0, 4), (0, 0)))
    wfc = wfc.reshape(_H, _XROW, _C2, _NCLS)[:, :_H]
    wfc = wfc.reshape(_KF, _NCLS)

    out = pl.pallas_call(
        _fused_body,
        out_shape=jax.ShapeDtypeStruct((n_pad, _NCLS), f32),
        grid=(n_pad // _S,),
        in_specs=[
            pl.BlockSpec((_S, _H * _H), lambda i: (i, 0)),
            pl.BlockSpec((_K1, _N1), lambda i: (0, 0)),
            pl.BlockSpec((1, _N1), lambda i: (0, 0)),
            pl.BlockSpec((_K2, _N2), lambda i: (0, 0)),
            pl.BlockSpec((1, _N2), lambda i: (0, 0)),
            pl.BlockSpec((_KF, _NCLS), lambda i: (0, 0)),
            pl.BlockSpec((1, _NCLS), lambda i: (0, 0)),
        ],
        out_specs=pl.BlockSpec((_S, _NCLS), lambda i: (i, 0)),
        scratch_shapes=[
            pltpu.VMEM((_S, _XROW * _XROW), f32),
            pltpu.VMEM((_S, 30 * _AROW), f32),
            pltpu.VMEM((_S, _KF), f32),
        ],
        compiler_params=pltpu.CompilerParams(
            dimension_semantics=("parallel",),
        ),
    )(x, w1r, b1row, w2r, b2row, wfc, bfc_row.astype(f32))
    return out[:n]


# speedup vs baseline: 19.5816x; 1.1573x over previous
import numpy as np

import jax
import jax.numpy as jnp
from jax.experimental import pallas as pl
from jax.experimental.pallas import tpu as pltpu

_S = 256
_H = 28
_C1, _C2, _NCLS = 10, 15, 10
_XROW = 32
_AROW = 320
_K1 = 6 * _XROW
_N1 = 2 * _H * _C1
_K2 = 2 * _AROW + 30 * _C1
_N2 = _H * _C2
_KF = _H * _H * _C2


_D1R = (np.arange(6)[None, :, None] - np.arange(2)[None, None, :]
        == np.arange(5)[:, None, None]).astype(np.float32)
_D1W = (np.arange(_XROW)[None, :, None] - np.arange(_H)[None, None, :]
        == np.arange(5)[:, None, None]).astype(np.float32)
_D1 = (_D1R[:, None, :, None, :, None] *
       _D1W[None, :, None, :, None, :]).reshape(25, 6, _XROW, 2, _H)
_D1 = np.ascontiguousarray(np.moveaxis(_D1, 0, -1)).reshape(-1, 25)
_dr = np.arange(3)[:, None, None, None, None, None]
_kw = np.arange(3)[None, :, None, None, None, None]
_b1 = np.arange(_C1)[None, None, :, None, None, None]
_pc = np.arange(_XROW)[None, None, None, :, None, None]
_b2 = np.arange(_C1)[None, None, None, None, :, None]
_w = np.arange(_H)[None, None, None, None, None, :]
_D2 = np.broadcast_to(
    (_pc - _w == _kw) & (_b1 == _b2),
    (3, 3, _C1, _XROW, _C1, _H)).astype(np.float32)
_D2 = np.ascontiguousarray(np.transpose(_D2, (0, 3, 4, 5, 1, 2)))
_D2 = _D2.reshape(3, _XROW, _C1, _H, 3 * _C1)


def _fused_body(x_ref, w1_ref, b1_ref, w2_ref, b2_ref, wfc_ref, bfc_ref,
                o_ref, xp_ref, a1_ref, a2_ref):
    f32 = jnp.float32
    bf16 = jnp.bfloat16

    xp_ref[...] = jnp.zeros((_S, _XROW * _XROW), bf16)
    for r in range(_H):
        base = _XROW * (r + 2) + 2
        xp_ref[:, base:base + _H] = x_ref[:, _H * r:_H * (r + 1)].astype(bf16)

    a1_ref[...] = jnp.zeros((_S, 30 * _AROW), bf16)

    b1v = b1_ref[...]
    for h in range(0, _H, 2):
        t = jnp.dot(xp_ref[:, _XROW * h:_XROW * h + _K1], w1_ref[...],
                    preferred_element_type=f32)
        t = jnp.maximum(t + b1v, 0.0).astype(bf16)
        a1_ref[:, _AROW * (h + 1) + 10:_AROW * (h + 1) + 290] = t[:, :280]
        a1_ref[:, _AROW * (h + 2) + 10:_AROW * (h + 2) + 290] = t[:, 280:]

    b2v = b2_ref[...]
    for h in range(_H):
        t = jnp.dot(a1_ref[:, _AROW * h:_AROW * h + _K2], w2_ref[...],
                    preferred_element_type=f32)
        a2_ref[:, _N2 * h:_N2 * (h + 1)] = jnp.maximum(t + b2v, 0.0).astype(bf16)

    logits = jnp.dot(a2_ref[...], wfc_ref[...], preferred_element_type=f32)
    o_ref[...] = (logits + bfc_ref[...]).astype(o_ref.dtype)


def kernel(w1p, b1c, w2p, b2c, wfc_canon, bfc_row, gmask, bdmask, fold, x_nchw):
    del gmask, bdmask, fold
    f32 = jnp.float32
    n = x_nchw.shape[0]
    x = x_nchw.astype(f32).reshape(n, _H * _H)
    n_pad = pl.cdiv(n, _S) * _S
    if n_pad != n:
        x = jnp.pad(x, ((0, n_pad - n), (0, 0)))

    bf16 = jnp.bfloat16
    w1r = jnp.dot(_D1, w1p.astype(f32).T)
    w1r = w1r.reshape(_K1, _N1).astype(bf16)
    b1row = jnp.tile(b1c.astype(f32)[:, 0], (2 * _H,)).reshape(1, _N1)
    w2p3 = w2p.astype(f32).reshape(_C2, 3, 3 * _C1)
    w2r = jnp.einsum('dpbwj,cdj->dpbwc', _D2, w2p3)
    w2r = w2r.reshape(3 * _XROW * _C1, _N2)[:_K2].astype(bf16)
    b2row = jnp.tile(b2c.astype(f32)[:, 0], (_H,)).reshape(1, _N2)
    wfc = jnp.pad(wfc_canon.astype(f32), ((0, 4), (0, 0)))
    wfc = wfc.reshape(_H, _XROW, _C2, _NCLS)[:, :_H]
    wfc = wfc.reshape(_KF, _NCLS).astype(bf16)

    out = pl.pallas_call(
        _fused_body,
        out_shape=jax.ShapeDtypeStruct((n_pad, _NCLS), f32),
        grid=(n_pad // _S,),
        in_specs=[
            pl.BlockSpec((_S, _H * _H), lambda i: (i, 0)),
            pl.BlockSpec((_K1, _N1), lambda i: (0, 0)),
            pl.BlockSpec((1, _N1), lambda i: (0, 0)),
            pl.BlockSpec((_K2, _N2), lambda i: (0, 0)),
            pl.BlockSpec((1, _N2), lambda i: (0, 0)),
            pl.BlockSpec((_KF, _NCLS), lambda i: (0, 0)),
            pl.BlockSpec((1, _NCLS), lambda i: (0, 0)),
        ],
        out_specs=pl.BlockSpec((_S, _NCLS), lambda i: (i, 0)),
        scratch_shapes=[
            pltpu.VMEM((_S, _XROW * _XROW), jnp.bfloat16),
            pltpu.VMEM((_S, 30 * _AROW), jnp.bfloat16),
            pltpu.VMEM((_S, _KF), jnp.bfloat16),
        ],
        compiler_params=pltpu.CompilerParams(
            dimension_semantics=("parallel",),
        ),
    )(x, w1r, b1row, w2r, b2row, wfc, bfc_row.astype(f32))
    return out[:n]
